# baseline (speedup 1.0000x reference)
import functools

import jax
import jax.numpy as jnp
from jax import lax
from jax.experimental import pallas as pl
from jax.experimental.pallas import tpu as pltpu
from jax.experimental.pallas import tpu_sc as plsc

_B = 4096
_T = 200
_V = 1000000
_D = 16
_BOS = 2

_NC = 2
_NS = 16
_NW = _NC * _NS
_VT = (_V + 127) // 128
_LAST_VT = _VT - 1
_LAST_W = _V - _LAST_VT * 128
_R = 6
_G = 3
_NCH = _T - 1


def _relayout_body(tab_ref, out_ref, in_v, out_v, isem, osem):
    wid = lax.axis_index("s") * _NC + lax.axis_index("c")
    iot = lax.iota(jnp.int32, 16)

    def _in(i, s):
        vt = wid + _NW * i
        return pltpu.make_async_copy(
            tab_ref.at[:, pl.ds(vt * 128, 128)], in_v.at[s], isem.at[s])

    def _in_last(s):
        return pltpu.make_async_copy(
            tab_ref.at[:, pl.ds(_LAST_VT * 128, _LAST_W)],
            in_v.at[s, :, pl.ds(0, _LAST_W)], isem.at[s])

    n_i = (_VT - wid + _NW - 1) // _NW

    def _fire_in(i):
        @pl.when(i < n_i)
        def _():
            s = lax.rem(i, 2)
            vt = wid + _NW * i

            @pl.when(vt == _LAST_VT)
            def _():
                _in_last(s).start()

            @pl.when(vt != _LAST_VT)
            def _():
                _in(i, s).start()

    _fire_in(jnp.int32(0))
    _fire_in(jnp.int32(1))

    def _step(i, carry):
        s = lax.rem(i, 2)
        vt = wid + _NW * i

        @pl.when(vt == _LAST_VT)
        def _():
            _in_last(s).wait()

        @pl.when(vt != _LAST_VT)
        def _():
            _in(i, s).wait()

        @pl.when(i >= 2)
        def _():
            vtp = wid + _NW * (i - 2)

            @pl.when(vtp == _LAST_VT)
            def _():
                pltpu.make_async_copy(
                    out_v.at[s, pl.ds(0, _LAST_W * 16)],
                    out_ref.at[pl.ds(vtp * 2048, _LAST_W * 16)],
                    osem.at[s]).wait()

            @pl.when(vtp != _LAST_VT)
            def _():
                pltpu.make_async_copy(
                    out_v.at[s], out_ref.at[pl.ds(vtp * 2048, 2048)],
                    osem.at[s]).wait()

        for j in range(128):
            out_v[s, pl.ds(j * 16, 16)] = plsc.load_gather(
                in_v.at[s], [iot, jnp.full((16,), j, jnp.int32)])

        @pl.when(vt == _LAST_VT)
        def _():
            pltpu.make_async_copy(
                out_v.at[s, pl.ds(0, _LAST_W * 16)],
                out_ref.at[pl.ds(vt * 2048, _LAST_W * 16)],
                osem.at[s]).start()

        @pl.when(vt != _LAST_VT)
        def _():
            pltpu.make_async_copy(
                out_v.at[s], out_ref.at[pl.ds(vt * 2048, 2048)],
                osem.at[s]).start()

        _fire_in(i + 2)
        return carry

    lax.fori_loop(0, n_i, _step, None)

    def _drain(i, carry):
        @pl.when(i >= n_i - 2)
        def _():
            s = lax.rem(i, 2)
            vt = wid + _NW * i

            @pl.when(vt == _LAST_VT)
            def _():
                pltpu.make_async_copy(
                    out_v.at[s, pl.ds(0, _LAST_W * 16)],
                    out_ref.at[pl.ds(vt * 2048, _LAST_W * 16)],
                    osem.at[s]).wait()

            @pl.when(vt != _LAST_VT)
            def _():
                pltpu.make_async_copy(
                    out_v.at[s], out_ref.at[pl.ds(vt * 2048, 2048)],
                    osem.at[s]).wait()

        return carry

    lax.fori_loop(jnp.maximum(n_i - 2, 0), n_i, _drain, None)


def _gather_body(idx_ref, tab_ref, out_ref, idx_v, rows_v, dblk, bosb, bos_i,
                 isem, gsem, wsem, sem0):
    wid = lax.axis_index("s") * _NC + lax.axis_index("c")
    iot = lax.iota(jnp.int32, 16)

    bos_i[pl.ds(0, 16)] = jnp.full((16,), _BOS, jnp.int32)
    pltpu.async_copy(tab_ref.at[bos_i], bosb, sem0).wait()
    for d in range(16):
        val = plsc.load_gather(bosb, [iot, jnp.full((16,), d, jnp.int32)])
        for j in range(8):
            dblk[0, d // 8, d % 8, pl.ds(j * 16, 16)] = val
    pltpu.async_copy(dblk.at[0, 0], out_ref.at[0, 0, wid], sem0).wait()
    pltpu.async_copy(dblk.at[0, 1], out_ref.at[0, 1, wid], sem0).wait()

    def _idx(tk, s):
        return pltpu.make_async_copy(idx_ref.at[tk, wid], idx_v.at[s],
                                     isem.at[s])

    _idx(jnp.int32(0), jnp.int32(0)).start()
    _idx(jnp.int32(1), jnp.int32(1)).start()

    def _gather(c, s):
        tk = lax.div(c, 8)
        tr = lax.rem(c, 8)
        return pltpu.make_async_copy(
            tab_ref.at[idx_v.at[lax.rem(tk, 2), tr]], rows_v.at[s],
            gsem.at[s])

    def _wr(c, s, dt):
        return pltpu.make_async_copy(
            dblk.at[s, dt], out_ref.at[c + 1, dt, wid], wsem.at[s])

    def _step(c, carry):
        s = lax.rem(c, _R)
        tr = lax.rem(c, 8)

        @pl.when(tr == 0)
        def _():
            tk = lax.div(c, 8)
            _idx(tk, lax.rem(tk, 2)).wait()

        _gather(c, s).start()

        @pl.when(c >= _G)
        def _():
            c2 = c - _G
            s2 = lax.rem(c2, _R)
            _gather(c2, s2).wait()

            @pl.when(tr == 2)
            def _():
                h = lax.div(c, 8)

                @pl.when((h >= 1) & (h + 1 < 25))
                def _():
                    _idx(h + 1, lax.rem(h + 1, 2)).start()

            @pl.when(c2 >= _R)
            def _():
                _wr(c2 - _R, s2, 0).wait()
                _wr(c2 - _R, s2, 1).wait()

            for d in range(16):
                for j in range(8):
                    dblk[s2, d // 8, d % 8, pl.ds(j * 16, 16)] = (
                        plsc.load_gather(
                            rows_v.at[s2],
                            [iot + 16 * j, jnp.full((16,), d, jnp.int32)]))
            _wr(c2, s2, 0).start()
            _wr(c2, s2, 1).start()

        return carry

    lax.fori_loop(0, _NCH, _step, None)

    for c2 in range(_NCH - _G, _NCH):
        s2 = c2 % _R
        _gather(jnp.int32(c2), s2).wait()
        _wr(c2 - _R, s2, 0).wait()
        _wr(c2 - _R, s2, 1).wait()
        for d in range(16):
            for j in range(8):
                dblk[s2, d // 8, d % 8, pl.ds(j * 16, 16)] = (
                    plsc.load_gather(
                        rows_v.at[s2],
                        [iot + 16 * j, jnp.full((16,), d, jnp.int32)]))
        _wr(jnp.int32(c2), s2, 0).start()
        _wr(jnp.int32(c2), s2, 1).start()
    for c2 in range(_NCH - _R, _NCH):
        s2 = c2 % _R
        _wr(jnp.int32(c2), s2, 0).wait()
        _wr(jnp.int32(c2), s2, 1).wait()


@jax.jit
def kernel(inputs, vocab_table):
    mesh = plsc.VectorSubcoreMesh(core_axis_name="c", subcore_axis_name="s")

    relayout = pl.kernel(
        _relayout_body,
        out_type=jax.ShapeDtypeStruct((_V * _D,), jnp.float32),
        mesh=mesh,
        compiler_params=pltpu.CompilerParams(use_tc_tiling_on_sc=False,
                                             needs_layout_passes=False),
        scratch_types=[
            pltpu.VMEM((2, 16, 128), jnp.float32),
            pltpu.VMEM((2, 2048), jnp.float32),
            pltpu.SemaphoreType.DMA((2,)),
            pltpu.SemaphoreType.DMA((2,)),
        ],
    )
    tflat = relayout(vocab_table.T)
    table2 = tflat.reshape(_V, _D)

    idx2 = inputs.T.reshape(25, 8, 32, 128).transpose(0, 2, 1, 3)

    gather = pl.kernel(
        _gather_body,
        out_type=jax.ShapeDtypeStruct((_T, 2, 32, 8, 128), jnp.float32),
        mesh=mesh,
        compiler_params=pltpu.CompilerParams(use_tc_tiling_on_sc=False,
                                             needs_layout_passes=False),
        scratch_types=[
            pltpu.VMEM((2, 8, 128), jnp.int32),
            pltpu.VMEM((_R, 128, 16), jnp.float32),
            pltpu.VMEM((_R, 2, 8, 128), jnp.float32),
            pltpu.VMEM((16, 16), jnp.float32),
            pltpu.VMEM((16,), jnp.int32),
            pltpu.SemaphoreType.DMA((2,)),
            pltpu.SemaphoreType.DMA((_R,)),
            pltpu.SemaphoreType.DMA((_R,)),
            pltpu.SemaphoreType.DMA,
        ],
    )
    out5 = gather(idx2, table2)
    return (out5.transpose(0, 1, 3, 2, 4).reshape(_T, _D, _B)
            .transpose(2, 0, 1))


# speedup vs baseline: 3.5708x; 3.5708x over previous
import functools

import jax
import jax.numpy as jnp
from jax import lax
from jax.experimental import pallas as pl
from jax.experimental.pallas import tpu as pltpu
from jax.experimental.pallas import tpu_sc as plsc

_B = 4096
_T = 200
_V = 1000000
_D = 16
_BOS = 2

_NC = 2
_NS = 16
_NW = _NC * _NS
_VT = (_V + 127) // 128
_LAST_VT = _VT - 1
_LAST_W = _V - _LAST_VT * 128
_R = 6
_G = 3
_NCH = _T - 1


def _relayout_body(tab_ref, out_ref, in_v, out_v, isem, osem):
    wid = lax.axis_index("s") * _NC + lax.axis_index("c")
    iot = lax.iota(jnp.int32, 16)

    def _in(i, s):
        vt = wid + _NW * i
        return pltpu.make_async_copy(
            tab_ref.at[:, pl.ds(vt * 128, 128)], in_v.at[s], isem.at[s])

    def _in_last(s):
        return pltpu.make_async_copy(
            tab_ref.at[:, pl.ds(_LAST_VT * 128, _LAST_W)],
            in_v.at[s, :, pl.ds(0, _LAST_W)], isem.at[s])

    n_i = (_VT - wid + _NW - 1) // _NW

    def _fire_in(i):
        @pl.when(i < n_i)
        def _():
            s = lax.rem(i, 2)
            vt = wid + _NW * i

            @pl.when(vt == _LAST_VT)
            def _():
                _in_last(s).start()

            @pl.when(vt != _LAST_VT)
            def _():
                _in(i, s).start()

    _fire_in(jnp.int32(0))
    _fire_in(jnp.int32(1))

    def _step(i, carry):
        s = lax.rem(i, 2)
        vt = wid + _NW * i

        @pl.when(vt == _LAST_VT)
        def _():
            _in_last(s).wait()

        @pl.when(vt != _LAST_VT)
        def _():
            _in(i, s).wait()

        @pl.when(i >= 2)
        def _():
            vtp = wid + _NW * (i - 2)

            @pl.when(vtp == _LAST_VT)
            def _():
                pltpu.make_async_copy(
                    out_v.at[s, pl.ds(0, _LAST_W * 16)],
                    out_ref.at[pl.ds(vtp * 2048, _LAST_W * 16)],
                    osem.at[s]).wait()

            @pl.when(vtp != _LAST_VT)
            def _():
                pltpu.make_async_copy(
                    out_v.at[s], out_ref.at[pl.ds(vtp * 2048, 2048)],
                    osem.at[s]).wait()

        for j in range(128):
            out_v[s, pl.ds(j * 16, 16)] = plsc.load_gather(
                in_v.at[s], [iot, jnp.full((16,), j, jnp.int32)])

        @pl.when(vt == _LAST_VT)
        def _():
            pltpu.make_async_copy(
                out_v.at[s, pl.ds(0, _LAST_W * 16)],
                out_ref.at[pl.ds(vt * 2048, _LAST_W * 16)],
                osem.at[s]).start()

        @pl.when(vt != _LAST_VT)
        def _():
            pltpu.make_async_copy(
                out_v.at[s], out_ref.at[pl.ds(vt * 2048, 2048)],
                osem.at[s]).start()

        _fire_in(i + 2)
        return carry

    lax.fori_loop(0, n_i, _step, None)

    def _drain(i, carry):
        @pl.when(i >= n_i - 2)
        def _():
            s = lax.rem(i, 2)
            vt = wid + _NW * i

            @pl.when(vt == _LAST_VT)
            def _():
                pltpu.make_async_copy(
                    out_v.at[s, pl.ds(0, _LAST_W * 16)],
                    out_ref.at[pl.ds(vt * 2048, _LAST_W * 16)],
                    osem.at[s]).wait()

            @pl.when(vt != _LAST_VT)
            def _():
                pltpu.make_async_copy(
                    out_v.at[s], out_ref.at[pl.ds(vt * 2048, 2048)],
                    osem.at[s]).wait()

        return carry

    lax.fori_loop(jnp.maximum(n_i - 2, 0), n_i, _drain, None)


def _gather_body(idx_ref, tab_ref, out_ref, idx_v, rows_v, dblk, bosb, bos_i,
                 isem, gsem, wsem, sem0):
    wid = lax.axis_index("s") * _NC + lax.axis_index("c")
    iot = lax.iota(jnp.int32, 16)

    bos_i[pl.ds(0, 16)] = jnp.full((16,), _BOS, jnp.int32)
    pltpu.async_copy(tab_ref.at[bos_i], bosb, sem0).wait()
    for d in range(16):
        val = plsc.load_gather(bosb, [iot, jnp.full((16,), d, jnp.int32)])
        for j in range(8):
            dblk[0, d // 8, d % 8, pl.ds(j * 16, 16)] = val
    pltpu.async_copy(dblk.at[0, 0], out_ref.at[0, 0, wid], sem0).wait()
    pltpu.async_copy(dblk.at[0, 1], out_ref.at[0, 1, wid], sem0).wait()

    def _idx(tk, s):
        return pltpu.make_async_copy(idx_ref.at[tk, wid], idx_v.at[s],
                                     isem.at[s])

    _idx(jnp.int32(0), jnp.int32(0)).start()
    _idx(jnp.int32(1), jnp.int32(1)).start()

    def _gather(c, s):
        tk = lax.div(c, 8)
        tr = lax.rem(c, 8)
        return pltpu.make_async_copy(
            tab_ref.at[idx_v.at[lax.rem(tk, 2), tr]], rows_v.at[s],
            gsem.at[s])

    def _wr(c, s, dt):
        return pltpu.make_async_copy(
            dblk.at[s, dt], out_ref.at[c + 1, dt, wid], wsem.at[s])

    def _step(c, carry):
        s = lax.rem(c, _R)
        tr = lax.rem(c, 8)

        @pl.when(tr == 0)
        def _():
            tk = lax.div(c, 8)
            _idx(tk, lax.rem(tk, 2)).wait()

        _gather(c, s).start()

        @pl.when(c >= _G)
        def _():
            c2 = c - _G
            s2 = lax.rem(c2, _R)
            _gather(c2, s2).wait()

            @pl.when(tr == 2)
            def _():
                h = lax.div(c, 8)

                @pl.when((h >= 1) & (h + 1 < 25))
                def _():
                    _idx(h + 1, lax.rem(h + 1, 2)).start()

            @pl.when(c2 >= _R)
            def _():
                _wr(c2 - _R, s2, 0).wait()
                _wr(c2 - _R, s2, 1).wait()

            for d in range(16):
                for j in range(8):
                    dblk[s2, d // 8, d % 8, pl.ds(j * 16, 16)] = (
                        plsc.load_gather(
                            rows_v.at[s2],
                            [iot + 16 * j, jnp.full((16,), d, jnp.int32)]))
            _wr(c2, s2, 0).start()
            _wr(c2, s2, 1).start()

        return carry

    lax.fori_loop(0, _NCH, _step, None)

    for c2 in range(_NCH - _G, _NCH):
        s2 = c2 % _R
        _gather(jnp.int32(c2), s2).wait()
        _wr(c2 - _R, s2, 0).wait()
        _wr(c2 - _R, s2, 1).wait()
        for d in range(16):
            for j in range(8):
                dblk[s2, d // 8, d % 8, pl.ds(j * 16, 16)] = (
                    plsc.load_gather(
                        rows_v.at[s2],
                        [iot + 16 * j, jnp.full((16,), d, jnp.int32)]))
        _wr(jnp.int32(c2), s2, 0).start()
        _wr(jnp.int32(c2), s2, 1).start()
    for c2 in range(_NCH - _R, _NCH):
        s2 = c2 % _R
        _wr(jnp.int32(c2), s2, 0).wait()
        _wr(jnp.int32(c2), s2, 1).wait()


def _tc_transpose_body(x_ref, o_ref):
    x = x_ref[...]
    y = x.T.reshape(1024, 8, 16)
    o_ref[...] = jnp.concatenate([y[:, h, :] for h in range(8)], axis=1)


@jax.jit
def kernel(inputs, vocab_table):
    mesh = plsc.VectorSubcoreMesh(core_axis_name="c", subcore_axis_name="s")

    tflat = pl.pallas_call(
        _tc_transpose_body,
        grid=(123,),
        in_specs=[pl.BlockSpec((16, 8192), lambda i: (0, i))],
        out_specs=pl.BlockSpec((1024, 128), lambda i: (i, 0)),
        out_shape=jax.ShapeDtypeStruct((_V * _D // 128, 128), jnp.float32),
    )(vocab_table.T)
    table2 = tflat.reshape(_V * _D).reshape(_V, _D)

    idx2 = inputs.T.reshape(25, 8, 32, 128).transpose(0, 2, 1, 3)

    gather = pl.kernel(
        _gather_body,
        out_type=jax.ShapeDtypeStruct((_T, 2, 32, 8, 128), jnp.float32),
        mesh=mesh,
        compiler_params=pltpu.CompilerParams(use_tc_tiling_on_sc=False,
                                             needs_layout_passes=False),
        scratch_types=[
            pltpu.VMEM((2, 8, 128), jnp.int32),
            pltpu.VMEM((_R, 128, 16), jnp.float32),
            pltpu.VMEM((_R, 2, 8, 128), jnp.float32),
            pltpu.VMEM((16, 16), jnp.float32),
            pltpu.VMEM((16,), jnp.int32),
            pltpu.SemaphoreType.DMA((2,)),
            pltpu.SemaphoreType.DMA((_R,)),
            pltpu.SemaphoreType.DMA((_R,)),
            pltpu.SemaphoreType.DMA,
        ],
    )
    out5 = gather(idx2, table2)
    return (out5.transpose(0, 1, 3, 2, 4).reshape(_T, _D, _B)
            .transpose(2, 0, 1))
